# Initial kernel scaffold; baseline (speedup 1.0000x reference)
import jax
import jax.numpy as jnp
from jax.experimental import pallas as pl


def kernel(x, edge_index, batch, edge_attr, labels, weight, W_emb, Wqk, Wv, Wout, bout, W1, b1, W2, b2):
    raise NotImplementedError("write your pallas kernel here")


# speedup vs baseline: 24.7941x; 24.7941x over previous
import functools
import jax
import jax.numpy as jnp
from jax import lax
from jax.experimental import pallas as pl
from jax.experimental.pallas import tpu as pltpu, tpu_sc as plsc

N = 10000
E = 320000
D = 128
H = 8
HD = 16
FF = 512
G = 16
SCALE = HD ** (-0.5)
ME_W = 144

NC, NS = 2, 16
NW = NC * NS
CH = 80
NCHUNK = 125
EPW = CH * NCHUNK
NPAD = 10240
RPT = NPAD // NS

F32 = jnp.float32


def _embed_body(x_ref, w_ref, o_ref):
    o_ref[...] = jnp.dot(x_ref[...], w_ref[...], preferred_element_type=F32)


_embed = pl.pallas_call(
    _embed_body,
    grid=(10,),
    in_specs=[
        pl.BlockSpec((1000, 100), lambda i: (i, 0)),
        pl.BlockSpec((100, D), lambda i: (0, 0)),
    ],
    out_specs=pl.BlockSpec((1000, D), lambda i: (i, 0)),
    out_shape=jax.ShapeDtypeStruct((N, D), F32),
)


def _qkv_body(h_ref, wqk_ref, wv_ref, qv_ref, k_ref):
    hb = h_ref[...]
    qk = jnp.dot(hb, wqk_ref[...], preferred_element_type=F32)
    qv_ref[:, :D] = qk[:, :D]
    qv_ref[:, D:] = jnp.dot(hb, wv_ref[...], preferred_element_type=F32)
    k_ref[...] = qk[:, D:]


_qkv = pl.pallas_call(
    _qkv_body,
    grid=(10,),
    in_specs=[
        pl.BlockSpec((1000, D), lambda i: (i, 0)),
        pl.BlockSpec((D, 2 * D), lambda i: (0, 0)),
        pl.BlockSpec((D, D), lambda i: (0, 0)),
    ],
    out_specs=[
        pl.BlockSpec((1000, 2 * D), lambda i: (i, 0)),
        pl.BlockSpec((1000, D), lambda i: (i, 0)),
    ],
    out_shape=[
        jax.ShapeDtypeStruct((N, 2 * D), F32),
        jax.ShapeDtypeStruct((N, D), F32),
    ],
)


def _edge_body(qvs_ref, kd_ref, me_ref):
    qvs = qvs_ref[...]
    q = qvs[:, :D]
    v = qvs[:, D:]
    prod = q * kd_ref[...]
    S = (lax.broadcasted_iota(jnp.int32, (D, H), 0) // HD
         == lax.broadcasted_iota(jnp.int32, (D, H), 1)).astype(F32)
    ea = jnp.exp(jnp.dot(prod, S, preferred_element_type=F32) * SCALE)
    T = (lax.broadcasted_iota(jnp.int32, (H, D), 1) // HD
         == lax.broadcasted_iota(jnp.int32, (H, D), 0)).astype(F32)
    msg = v * jnp.dot(ea, T, preferred_element_type=F32)
    J = (lax.broadcasted_iota(jnp.int32, (H, 16), 1)
         == lax.broadcasted_iota(jnp.int32, (H, 16), 0)).astype(F32)
    me_ref[...] = jnp.concatenate(
        [msg, jnp.dot(ea, J, preferred_element_type=F32)], axis=-1)


_edge = pl.pallas_call(
    _edge_body,
    grid=(160,),
    in_specs=[
        pl.BlockSpec((2000, 2 * D), lambda i: (i, 0)),
        pl.BlockSpec((2000, D), lambda i: (i, 0)),
    ],
    out_specs=pl.BlockSpec((2000, ME_W), lambda i: (i, 0)),
    out_shape=jax.ShapeDtypeStruct((E, ME_W), F32),
)


def _ffn_body(h_ref, acc_ref, wout_ref, bout_ref, w1_ref, b1_ref, w2_ref,
              b2_ref, o_ref):
    accs = acc_ref[0] + acc_ref[1]
    numer = accs[:, :D]
    P = (lax.broadcasted_iota(jnp.int32, (16, D), 1) // HD
         == lax.broadcasted_iota(jnp.int32, (16, D), 0)).astype(F32)
    den = jnp.dot(accs[:, D:], P, preferred_element_type=F32)
    att = numer / (den + 1e-16)
    h1 = (h_ref[...] + jnp.dot(att, wout_ref[...], preferred_element_type=F32)
          + bout_ref[...])
    ff = jnp.maximum(
        jnp.dot(h1, w1_ref[...], preferred_element_type=F32) + b1_ref[...], 0.0)
    o_ref[...] = h1 + jnp.dot(ff, w2_ref[...], preferred_element_type=F32) + b2_ref[...]


_ffn = pl.pallas_call(
    _ffn_body,
    grid=(10,),
    in_specs=[
        pl.BlockSpec((1000, D), lambda i: (i, 0)),
        pl.BlockSpec((2, 1000, ME_W), lambda i: (0, i, 0)),
        pl.BlockSpec((D, D), lambda i: (0, 0)),
        pl.BlockSpec((1, D), lambda i: (0, 0)),
        pl.BlockSpec((D, FF), lambda i: (0, 0)),
        pl.BlockSpec((1, FF), lambda i: (0, 0)),
        pl.BlockSpec((FF, D), lambda i: (0, 0)),
        pl.BlockSpec((1, D), lambda i: (0, 0)),
    ],
    out_specs=pl.BlockSpec((1000, D), lambda i: (i, 0)),
    out_shape=jax.ShapeDtypeStruct((N, D), F32),
)


def _pool_body(h_ref, b_ref, o_ref, s_acc, c_acc):
    i = pl.program_id(0)

    @pl.when(i == 0)
    def _():
        s_acc[...] = jnp.zeros_like(s_acc)
        c_acc[...] = jnp.zeros_like(c_acc)

    hb = h_ref[...]
    oh = (b_ref[...] == lax.broadcasted_iota(jnp.int32, (1, G), 1)).astype(F32)
    dn = (((0,), (0,)), ((), ()))
    s_acc[...] += lax.dot_general(oh, hb, dn, preferred_element_type=F32)
    c_acc[...] += lax.dot_general(oh, jnp.ones_like(hb), dn,
                                  preferred_element_type=F32)

    @pl.when(i == 9)
    def _():
        o_ref[...] = s_acc[...] / jnp.maximum(c_acc[...], 1.0)


_pool = pl.pallas_call(
    _pool_body,
    grid=(10,),
    in_specs=[
        pl.BlockSpec((1000, D), lambda i: (i, 0)),
        pl.BlockSpec((1000, 1), lambda i: (i, 0)),
    ],
    out_specs=pl.BlockSpec((G, D), lambda i: (0, 0)),
    out_shape=jax.ShapeDtypeStruct((G, D), F32),
    scratch_shapes=[pltpu.VMEM((G, D), F32), pltpu.VMEM((G, D), F32)],
)


@functools.lru_cache(maxsize=None)
def _sc_kernels():
    mesh = plsc.VectorSubcoreMesh(core_axis_name="c", subcore_axis_name="s",
                                  num_cores=NC, num_subcores=NS)

    @functools.partial(
        pl.kernel,
        out_type=[
            jax.ShapeDtypeStruct((E, 2 * D), F32),
            jax.ShapeDtypeStruct((E, D), F32),
        ],
        mesh=mesh,
        scratch_types=[
            pltpu.VMEM((CH,), jnp.int32),
            pltpu.VMEM((CH,), jnp.int32),
            pltpu.VMEM((CH, 2 * D), F32),
            pltpu.VMEM((CH, D), F32),
            pltpu.SemaphoreType.DMA,
        ],
    )
    def gather(qv_hbm, k_hbm, src_hbm, dst_hbm, qvs_hbm, kd_hbm,
               is_, id_, qb, kb, sem):
        wid = lax.axis_index("s") * NC + lax.axis_index("c")
        base_w = wid * EPW

        def body(i, carry):
            base = base_w + i * CH
            pltpu.sync_copy(src_hbm.at[pl.ds(base, CH)], is_)
            pltpu.sync_copy(dst_hbm.at[pl.ds(base, CH)], id_)
            pltpu.async_copy(qv_hbm.at[is_], qb, sem).wait()
            pltpu.async_copy(k_hbm.at[id_], kb, sem).wait()
            pltpu.sync_copy(qb, qvs_hbm.at[pl.ds(base, CH)])
            pltpu.sync_copy(kb, kd_hbm.at[pl.ds(base, CH)])
            return carry

        lax.fori_loop(0, NCHUNK, body, 0)

    @functools.partial(
        pl.kernel,
        out_type=jax.ShapeDtypeStruct((NC, NPAD, ME_W), F32),
        mesh=mesh,
        scratch_types=[
            pltpu.VMEM((CH,), jnp.int32),
            pltpu.VMEM((CH, ME_W), F32),
            pltpu.VMEM_SHARED((NPAD, ME_W), F32),
            pltpu.SemaphoreType.DMA,
        ],
        compiler_params=pltpu.CompilerParams(use_tc_tiling_on_sc=False),
    )
    def scatter(me_hbm, dst_hbm, zero_hbm, acc_hbm, id_, mb, accs, sem):
        cid = lax.axis_index("c")
        sid = lax.axis_index("s")
        wid = sid * NC + cid
        pltpu.sync_copy(zero_hbm, accs.at[pl.ds(sid * RPT, RPT)])
        plsc.subcore_barrier()

        def body(i, carry):
            base = wid * EPW + i * CH
            pltpu.sync_copy(dst_hbm.at[pl.ds(base, CH)], id_)
            pltpu.sync_copy(me_hbm.at[pl.ds(base, CH)], mb)
            pltpu.sync_copy(mb, accs.at[id_], add=True)
            return carry

        lax.fori_loop(0, NCHUNK, body, 0)
        plsc.subcore_barrier()
        pltpu.sync_copy(accs.at[pl.ds(sid * RPT, RPT)],
                        acc_hbm.at[cid, pl.ds(sid * RPT, RPT)])

    return gather, scatter


def kernel(x, edge_index, batch, edge_attr, labels, weight, W_emb, Wqk, Wv,
           Wout, bout, W1, b1, W2, b2):
    src = edge_index[0].astype(jnp.int32)
    dst = edge_index[1].astype(jnp.int32)
    zero = jnp.zeros((RPT, ME_W), F32)
    sc_gather, sc_scatter = _sc_kernels()
    h = _embed(x, W_emb)
    for l in range(2):
        qv, kk = _qkv(h, Wqk[l], Wv[l])
        qvs, kd = sc_gather(qv, kk, src, dst)
        me = _edge(qvs, kd)
        acc = sc_scatter(me, dst, zero)
        h = _ffn(h, acc, Wout[l], bout[l].reshape(1, D), W1[l],
                 b1[l].reshape(1, FF), W2[l], b2[l].reshape(1, D))
    return _pool(h, batch.astype(jnp.int32).reshape(N, 1))


# speedup vs baseline: 32.9486x; 1.3289x over previous
import functools
import jax
import jax.numpy as jnp
from jax import lax
from jax.experimental import pallas as pl
from jax.experimental.pallas import tpu as pltpu, tpu_sc as plsc

N = 10000
E = 320000
D = 128
H = 8
HD = 16
FF = 512
G = 16
SCALE = HD ** (-0.5)
ME_W = 144

NC, NS = 2, 16
NW = NC * NS
CH = 80
NCHUNK = 125
EPW = CH * NCHUNK
NPAD = 10240
RPT = NPAD // NS

F32 = jnp.float32


def _embed_body(x_ref, w_ref, o_ref):
    o_ref[...] = jnp.dot(x_ref[...], w_ref[...], preferred_element_type=F32)


_embed = pl.pallas_call(
    _embed_body,
    grid=(10,),
    in_specs=[
        pl.BlockSpec((1000, 100), lambda i: (i, 0)),
        pl.BlockSpec((100, D), lambda i: (0, 0)),
    ],
    out_specs=pl.BlockSpec((1000, D), lambda i: (i, 0)),
    out_shape=jax.ShapeDtypeStruct((N, D), F32),
)


def _qkv_body(h_ref, wqk_ref, wv_ref, qv_ref, k_ref):
    hb = h_ref[...]
    qk = jnp.dot(hb, wqk_ref[...], preferred_element_type=F32)
    qv_ref[:, :D] = qk[:, :D]
    qv_ref[:, D:] = jnp.dot(hb, wv_ref[...], preferred_element_type=F32)
    k_ref[...] = qk[:, D:]


_qkv = pl.pallas_call(
    _qkv_body,
    grid=(10,),
    in_specs=[
        pl.BlockSpec((1000, D), lambda i: (i, 0)),
        pl.BlockSpec((D, 2 * D), lambda i: (0, 0)),
        pl.BlockSpec((D, D), lambda i: (0, 0)),
    ],
    out_specs=[
        pl.BlockSpec((1000, 2 * D), lambda i: (i, 0)),
        pl.BlockSpec((1000, D), lambda i: (i, 0)),
    ],
    out_shape=[
        jax.ShapeDtypeStruct((N, 2 * D), F32),
        jax.ShapeDtypeStruct((N, D), F32),
    ],
)


def _edge_body(qvs_ref, kd_ref, me_ref):
    qvs = qvs_ref[...]
    q = qvs[:, :D]
    v = qvs[:, D:]
    prod = q * kd_ref[...]
    S = (lax.broadcasted_iota(jnp.int32, (D, H), 0) // HD
         == lax.broadcasted_iota(jnp.int32, (D, H), 1)).astype(F32)
    ea = jnp.exp(jnp.dot(prod, S, preferred_element_type=F32) * SCALE)
    T = (lax.broadcasted_iota(jnp.int32, (H, D), 1) // HD
         == lax.broadcasted_iota(jnp.int32, (H, D), 0)).astype(F32)
    msg = v * jnp.dot(ea, T, preferred_element_type=F32)
    J = (lax.broadcasted_iota(jnp.int32, (H, 16), 1)
         == lax.broadcasted_iota(jnp.int32, (H, 16), 0)).astype(F32)
    me_ref[...] = jnp.concatenate(
        [msg, jnp.dot(ea, J, preferred_element_type=F32)], axis=-1)


_edge = pl.pallas_call(
    _edge_body,
    grid=(160,),
    in_specs=[
        pl.BlockSpec((2000, 2 * D), lambda i: (i, 0)),
        pl.BlockSpec((2000, D), lambda i: (i, 0)),
    ],
    out_specs=pl.BlockSpec((2000, ME_W), lambda i: (i, 0)),
    out_shape=jax.ShapeDtypeStruct((E, ME_W), F32),
)


def _ffn_body(h_ref, acc_ref, wout_ref, bout_ref, w1_ref, b1_ref, w2_ref,
              b2_ref, o_ref):
    accs = acc_ref[0] + acc_ref[1]
    numer = accs[:, :D]
    P = (lax.broadcasted_iota(jnp.int32, (16, D), 1) // HD
         == lax.broadcasted_iota(jnp.int32, (16, D), 0)).astype(F32)
    den = jnp.dot(accs[:, D:], P, preferred_element_type=F32)
    att = numer / (den + 1e-16)
    h1 = (h_ref[...] + jnp.dot(att, wout_ref[...], preferred_element_type=F32)
          + bout_ref[...])
    ff = jnp.maximum(
        jnp.dot(h1, w1_ref[...], preferred_element_type=F32) + b1_ref[...], 0.0)
    o_ref[...] = h1 + jnp.dot(ff, w2_ref[...], preferred_element_type=F32) + b2_ref[...]


_ffn = pl.pallas_call(
    _ffn_body,
    grid=(10,),
    in_specs=[
        pl.BlockSpec((1000, D), lambda i: (i, 0)),
        pl.BlockSpec((2, 1000, ME_W), lambda i: (0, i, 0)),
        pl.BlockSpec((D, D), lambda i: (0, 0)),
        pl.BlockSpec((1, D), lambda i: (0, 0)),
        pl.BlockSpec((D, FF), lambda i: (0, 0)),
        pl.BlockSpec((1, FF), lambda i: (0, 0)),
        pl.BlockSpec((FF, D), lambda i: (0, 0)),
        pl.BlockSpec((1, D), lambda i: (0, 0)),
    ],
    out_specs=pl.BlockSpec((1000, D), lambda i: (i, 0)),
    out_shape=jax.ShapeDtypeStruct((N, D), F32),
)


def _pool_body(h_ref, b_ref, o_ref, s_acc, c_acc):
    i = pl.program_id(0)

    @pl.when(i == 0)
    def _():
        s_acc[...] = jnp.zeros_like(s_acc)
        c_acc[...] = jnp.zeros_like(c_acc)

    hb = h_ref[...]
    oh = (b_ref[...] == lax.broadcasted_iota(jnp.int32, (1, G), 1)).astype(F32)
    dn = (((0,), (0,)), ((), ()))
    s_acc[...] += lax.dot_general(oh, hb, dn, preferred_element_type=F32)
    c_acc[...] += lax.dot_general(oh, jnp.ones_like(hb), dn,
                                  preferred_element_type=F32)

    @pl.when(i == 9)
    def _():
        o_ref[...] = s_acc[...] / jnp.maximum(c_acc[...], 1.0)


_pool = pl.pallas_call(
    _pool_body,
    grid=(10,),
    in_specs=[
        pl.BlockSpec((1000, D), lambda i: (i, 0)),
        pl.BlockSpec((1000, 1), lambda i: (i, 0)),
    ],
    out_specs=pl.BlockSpec((G, D), lambda i: (0, 0)),
    out_shape=jax.ShapeDtypeStruct((G, D), F32),
    scratch_shapes=[pltpu.VMEM((G, D), F32), pltpu.VMEM((G, D), F32)],
)


@functools.lru_cache(maxsize=None)
def _sc_kernels():
    mesh = plsc.VectorSubcoreMesh(core_axis_name="c", subcore_axis_name="s",
                                  num_cores=NC, num_subcores=NS)

    @functools.partial(
        pl.kernel,
        out_type=[
            jax.ShapeDtypeStruct((E, 2 * D), F32),
            jax.ShapeDtypeStruct((E, D), F32),
        ],
        mesh=mesh,
        scratch_types=[
            pltpu.VMEM((NCHUNK, CH), jnp.int32),
            pltpu.VMEM((NCHUNK, CH), jnp.int32),
            pltpu.VMEM((CH, 2 * D), F32),
            pltpu.VMEM((CH, 2 * D), F32),
            pltpu.VMEM((CH, D), F32),
            pltpu.VMEM((CH, D), F32),
            pltpu.SemaphoreType.DMA,
            pltpu.SemaphoreType.DMA,
        ],
    )
    def gather(qv_hbm, k_hbm, src_hbm, dst_hbm, qvs_hbm, kd_hbm,
               is_, id_, qb0, qb1, kb0, kb1, sem_g, sem_w):
        wid = lax.axis_index("s") * NC + lax.axis_index("c")
        base_w = wid * EPW
        qb = (qb0, qb1)
        kb = (kb0, kb1)
        pltpu.sync_copy(src_hbm.at[wid], is_)
        pltpu.sync_copy(dst_hbm.at[wid], id_)

        def issue_gather(c, b):
            pltpu.async_copy(qv_hbm.at[is_.at[c]], qb[b], sem_g)
            pltpu.async_copy(k_hbm.at[id_.at[c]], kb[b], sem_g)

        def wait_gather(b):
            pltpu.make_async_copy(qv_hbm.at[pl.ds(0, CH)], qb[b], sem_g).wait()
            pltpu.make_async_copy(k_hbm.at[pl.ds(0, CH)], kb[b], sem_g).wait()

        def issue_wb(c, b):
            base = base_w + c * CH
            pltpu.async_copy(qb[b], qvs_hbm.at[pl.ds(base, CH)], sem_w)
            pltpu.async_copy(kb[b], kd_hbm.at[pl.ds(base, CH)], sem_w)

        def wait_wb(b):
            pltpu.make_async_copy(qb[b], qvs_hbm.at[pl.ds(0, CH)], sem_w).wait()
            pltpu.make_async_copy(kb[b], kd_hbm.at[pl.ds(0, CH)], sem_w).wait()

        issue_gather(0, 0)
        wait_gather(0)
        issue_gather(1, 1)
        issue_wb(0, 0)
        wait_gather(1)
        wait_wb(0)
        issue_gather(2, 0)
        issue_wb(1, 1)

        @pl.loop(0, (NCHUNK - 3) // 2)
        def _(o):
            for b in (0, 1):
                c = 2 * o + 2 + b
                wait_gather(b)
                wait_wb(1 - b)
                issue_gather(c + 1, 1 - b)
                issue_wb(c, b)

        wait_gather(0)
        wait_wb(1)
        issue_wb(NCHUNK - 1, 0)
        wait_wb(0)

    @functools.partial(
        pl.kernel,
        out_type=jax.ShapeDtypeStruct((NC, NPAD, ME_W), F32),
        mesh=mesh,
        scratch_types=[
            pltpu.VMEM((NCHUNK, CH), jnp.int32),
            pltpu.VMEM((CH, ME_W), F32),
            pltpu.VMEM((CH, ME_W), F32),
            pltpu.VMEM_SHARED((NPAD, ME_W), F32),
            pltpu.SemaphoreType.DMA,
        ],
        compiler_params=pltpu.CompilerParams(use_tc_tiling_on_sc=False),
    )
    def scatter(me_hbm, dst_hbm, zero_hbm, acc_hbm, id_, mb0, mb1, accs, sem):
        cid = lax.axis_index("c")
        sid = lax.axis_index("s")
        wid = sid * NC + cid
        mb = (mb0, mb1)
        pltpu.sync_copy(dst_hbm.at[wid], id_)
        pltpu.sync_copy(zero_hbm, accs.at[pl.ds(sid * RPT, RPT)])
        plsc.subcore_barrier()

        def issue_fetch(c, b):
            pltpu.async_copy(me_hbm.at[pl.ds(wid * EPW + c * CH, CH)],
                             mb[b], sem)

        def wait_fetch(b):
            pltpu.make_async_copy(me_hbm.at[pl.ds(0, CH)], mb[b], sem).wait()

        issue_fetch(0, 0)

        @pl.loop(0, (NCHUNK - 1) // 2)
        def _(o):
            for b in (0, 1):
                c = 2 * o + b
                wait_fetch(b)
                issue_fetch(c + 1, 1 - b)
                pltpu.sync_copy(mb[b], accs.at[id_.at[c]], add=True)

        wait_fetch(0)
        pltpu.sync_copy(mb[0], accs.at[id_.at[NCHUNK - 1]], add=True)
        plsc.subcore_barrier()
        pltpu.sync_copy(accs.at[pl.ds(sid * RPT, RPT)],
                        acc_hbm.at[cid, pl.ds(sid * RPT, RPT)])

    return gather, scatter


def kernel(x, edge_index, batch, edge_attr, labels, weight, W_emb, Wqk, Wv,
           Wout, bout, W1, b1, W2, b2):
    src = edge_index[0].astype(jnp.int32).reshape(NW, NCHUNK, CH)
    dst = edge_index[1].astype(jnp.int32).reshape(NW, NCHUNK, CH)
    zero = jnp.zeros((RPT, ME_W), F32)
    sc_gather, sc_scatter = _sc_kernels()
    h = _embed(x, W_emb)
    for l in range(2):
        qv, kk = _qkv(h, Wqk[l], Wv[l])
        qvs, kd = sc_gather(qv, kk, src, dst)
        me = _edge(qvs, kd)
        acc = sc_scatter(me, dst, zero)
        h = _ffn(h, acc, Wout[l], bout[l].reshape(1, D), W1[l],
                 b1[l].reshape(1, FF), W2[l], b2[l].reshape(1, D))
    return _pool(h, batch.astype(jnp.int32).reshape(N, 1))


# speedup vs baseline: 40.9980x; 1.2443x over previous
import functools
import jax
import jax.numpy as jnp
from jax import lax
from jax.experimental import pallas as pl
from jax.experimental.pallas import tpu as pltpu, tpu_sc as plsc

N = 10000
E = 320000
D = 128
H = 8
HD = 16
FF = 512
G = 16
SCALE = HD ** (-0.5)
ME_W = 144

NC, NS = 2, 16
NW = NC * NS
CH = 40
NCHUNK = 250
EPW = CH * NCHUNK
NPAD = 10240
RPT = NPAD // NS

F32 = jnp.float32


def _embed_body(x_ref, w_ref, o_ref):
    o_ref[...] = jnp.dot(x_ref[...], w_ref[...], preferred_element_type=F32)


_embed = pl.pallas_call(
    _embed_body,
    grid=(10,),
    in_specs=[
        pl.BlockSpec((1000, 100), lambda i: (i, 0)),
        pl.BlockSpec((100, D), lambda i: (0, 0)),
    ],
    out_specs=pl.BlockSpec((1000, D), lambda i: (i, 0)),
    out_shape=jax.ShapeDtypeStruct((N, D), F32),
)


def _qkv_body(h_ref, wqk_ref, wv_ref, qv_ref, k_ref):
    hb = h_ref[...]
    qk = jnp.dot(hb, wqk_ref[...], preferred_element_type=F32)
    qv_ref[:, :D] = qk[:, :D]
    qv_ref[:, D:] = jnp.dot(hb, wv_ref[...], preferred_element_type=F32)
    k_ref[...] = qk[:, D:]


_qkv = pl.pallas_call(
    _qkv_body,
    grid=(10,),
    in_specs=[
        pl.BlockSpec((1000, D), lambda i: (i, 0)),
        pl.BlockSpec((D, 2 * D), lambda i: (0, 0)),
        pl.BlockSpec((D, D), lambda i: (0, 0)),
    ],
    out_specs=[
        pl.BlockSpec((1000, 2 * D), lambda i: (i, 0)),
        pl.BlockSpec((1000, D), lambda i: (i, 0)),
    ],
    out_shape=[
        jax.ShapeDtypeStruct((N, 2 * D), F32),
        jax.ShapeDtypeStruct((N, D), F32),
    ],
)


def _edge_body(qvs_ref, kd_ref, me_ref):
    qvs = qvs_ref[...]
    q = qvs[:, :D]
    v = qvs[:, D:]
    prod = q * kd_ref[...]
    S = (lax.broadcasted_iota(jnp.int32, (D, H), 0) // HD
         == lax.broadcasted_iota(jnp.int32, (D, H), 1)).astype(F32)
    ea = jnp.exp(jnp.dot(prod, S, preferred_element_type=F32) * SCALE)
    T = (lax.broadcasted_iota(jnp.int32, (H, D), 1) // HD
         == lax.broadcasted_iota(jnp.int32, (H, D), 0)).astype(F32)
    msg = v * jnp.dot(ea, T, preferred_element_type=F32)
    J = (lax.broadcasted_iota(jnp.int32, (H, 16), 1)
         == lax.broadcasted_iota(jnp.int32, (H, 16), 0)).astype(F32)
    me_ref[...] = jnp.concatenate(
        [msg, jnp.dot(ea, J, preferred_element_type=F32)], axis=-1)


_edge = pl.pallas_call(
    _edge_body,
    grid=(160,),
    in_specs=[
        pl.BlockSpec((2000, 2 * D), lambda i: (i, 0)),
        pl.BlockSpec((2000, D), lambda i: (i, 0)),
    ],
    out_specs=pl.BlockSpec((2000, ME_W), lambda i: (i, 0)),
    out_shape=jax.ShapeDtypeStruct((E, ME_W), F32),
)


def _ffn_body(h_ref, acc_ref, wout_ref, bout_ref, w1_ref, b1_ref, w2_ref,
              b2_ref, o_ref):
    accs = acc_ref[0] + acc_ref[1]
    numer = accs[:, :D]
    P = (lax.broadcasted_iota(jnp.int32, (16, D), 1) // HD
         == lax.broadcasted_iota(jnp.int32, (16, D), 0)).astype(F32)
    den = jnp.dot(accs[:, D:], P, preferred_element_type=F32)
    att = numer / (den + 1e-16)
    h1 = (h_ref[...] + jnp.dot(att, wout_ref[...], preferred_element_type=F32)
          + bout_ref[...])
    ff = jnp.maximum(
        jnp.dot(h1, w1_ref[...], preferred_element_type=F32) + b1_ref[...], 0.0)
    o_ref[...] = h1 + jnp.dot(ff, w2_ref[...], preferred_element_type=F32) + b2_ref[...]


_ffn = pl.pallas_call(
    _ffn_body,
    grid=(10,),
    in_specs=[
        pl.BlockSpec((1000, D), lambda i: (i, 0)),
        pl.BlockSpec((2, 1000, ME_W), lambda i: (0, i, 0)),
        pl.BlockSpec((D, D), lambda i: (0, 0)),
        pl.BlockSpec((1, D), lambda i: (0, 0)),
        pl.BlockSpec((D, FF), lambda i: (0, 0)),
        pl.BlockSpec((1, FF), lambda i: (0, 0)),
        pl.BlockSpec((FF, D), lambda i: (0, 0)),
        pl.BlockSpec((1, D), lambda i: (0, 0)),
    ],
    out_specs=pl.BlockSpec((1000, D), lambda i: (i, 0)),
    out_shape=jax.ShapeDtypeStruct((N, D), F32),
)


def _pool_body(h_ref, b_ref, o_ref, s_acc, c_acc):
    i = pl.program_id(0)

    @pl.when(i == 0)
    def _():
        s_acc[...] = jnp.zeros_like(s_acc)
        c_acc[...] = jnp.zeros_like(c_acc)

    hb = h_ref[...]
    oh = (b_ref[...] == lax.broadcasted_iota(jnp.int32, (1, G), 1)).astype(F32)
    dn = (((0,), (0,)), ((), ()))
    s_acc[...] += lax.dot_general(oh, hb, dn, preferred_element_type=F32)
    c_acc[...] += lax.dot_general(oh, jnp.ones_like(hb), dn,
                                  preferred_element_type=F32)

    @pl.when(i == 9)
    def _():
        o_ref[...] = s_acc[...] / jnp.maximum(c_acc[...], 1.0)


_pool = pl.pallas_call(
    _pool_body,
    grid=(10,),
    in_specs=[
        pl.BlockSpec((1000, D), lambda i: (i, 0)),
        pl.BlockSpec((1000, 1), lambda i: (i, 0)),
    ],
    out_specs=pl.BlockSpec((G, D), lambda i: (0, 0)),
    out_shape=jax.ShapeDtypeStruct((G, D), F32),
    scratch_shapes=[pltpu.VMEM((G, D), F32), pltpu.VMEM((G, D), F32)],
)


@functools.lru_cache(maxsize=None)
def _sc_kernels():
    mesh = plsc.VectorSubcoreMesh(core_axis_name="c", subcore_axis_name="s",
                                  num_cores=NC, num_subcores=NS)

    @functools.partial(
        pl.kernel,
        out_type=jax.ShapeDtypeStruct((NC, NPAD, ME_W), F32),
        mesh=mesh,
        scratch_types=[
            pltpu.VMEM((CH,), jnp.int32),
            pltpu.VMEM((CH,), jnp.int32),
            pltpu.VMEM((CH,), jnp.int32),
            pltpu.VMEM((CH,), jnp.int32),
            pltpu.VMEM((CH, 2 * D), F32),
            pltpu.VMEM((CH, 2 * D), F32),
            pltpu.VMEM((CH, D), F32),
            pltpu.VMEM((CH, D), F32),
            pltpu.VMEM((CH, ME_W), F32),
            pltpu.VMEM((16, 16), F32),
            pltpu.VMEM_SHARED((NPAD, ME_W), F32),
            pltpu.SemaphoreType.DMA,
            pltpu.SemaphoreType.DMA,
        ],
        compiler_params=pltpu.CompilerParams(use_tc_tiling_on_sc=False,
                                            needs_layout_passes=False),
    )
    def fused(qv_hbm, k_hbm, src_hbm, dst_hbm, zero_hbm, cons_hbm, acc_hbm,
              is0, is1, id0, id1, qvb0, qvb1, kb0, kb1, mb, cbuf, accs,
              sem_i, sem_g):
        cid = lax.axis_index("c")
        sid = lax.axis_index("s")
        wid = sid * NC + cid
        isb = (is0, is1)
        idb = (id0, id1)
        qvb = (qvb0, qvb1)
        kb = (kb0, kb1)
        pltpu.sync_copy(cons_hbm, cbuf)
        pltpu.sync_copy(zero_hbm, accs.at[pl.ds(sid * RPT, RPT)])
        plsc.subcore_barrier()

        def issue_idx(c, b):
            pltpu.async_copy(src_hbm.at[wid, c], isb[b], sem_i)
            pltpu.async_copy(dst_hbm.at[wid, c], idb[b], sem_i)

        def wait_idx(b):
            pltpu.make_async_copy(src_hbm.at[0, 0], isb[b], sem_i).wait()
            pltpu.make_async_copy(dst_hbm.at[0, 0], idb[b], sem_i).wait()

        def issue_gather(b):
            pltpu.async_copy(qv_hbm.at[isb[b]], qvb[b], sem_g)
            pltpu.async_copy(k_hbm.at[idb[b]], kb[b], sem_g)

        def wait_gather(b):
            pltpu.make_async_copy(qv_hbm.at[pl.ds(0, CH)], qvb[b], sem_g).wait()
            pltpu.make_async_copy(k_hbm.at[pl.ds(0, CH)], kb[b], sem_g).wait()

        def compute_scatter(b):
            qvc = qvb[b]
            kc = kb[b]

            @pl.loop(0, CH)
            def _(e):
                p = qvc[e, pl.ds(0, 16)] * kc[e, pl.ds(0, 16)]
                ea_acc = cbuf[0] * jnp.sum(p)
                for h in range(1, H):
                    p = qvc[e, pl.ds(16 * h, 16)] * kc[e, pl.ds(16 * h, 16)]
                    ea_acc = ea_acc + cbuf[h] * jnp.sum(p)
                ea = jnp.exp(ea_acc * SCALE) * cbuf[H]
                mb[e, pl.ds(D, 16)] = ea
                for h in range(H):
                    mb[e, pl.ds(16 * h, 16)] = (
                        qvc[e, pl.ds(D + 16 * h, 16)] * ea[h])

            pltpu.sync_copy(mb, accs.at[idb[b]], add=True)

        issue_idx(0, 0)
        issue_idx(1, 1)
        wait_idx(0)
        issue_gather(0)

        @pl.loop(0, (NCHUNK - 2) // 2)
        def _(o):
            for b in (0, 1):
                c = 2 * o + b
                wait_gather(b)
                wait_idx(1 - b)
                issue_gather(1 - b)
                compute_scatter(b)
                issue_idx(c + 2, b)

        wait_gather(0)
        wait_idx(1)
        issue_gather(1)
        compute_scatter(0)
        wait_gather(1)
        compute_scatter(1)
        plsc.subcore_barrier()
        pltpu.sync_copy(accs.at[pl.ds(sid * RPT, RPT)],
                        acc_hbm.at[cid, pl.ds(sid * RPT, RPT)])

    return fused


def kernel(x, edge_index, batch, edge_attr, labels, weight, W_emb, Wqk, Wv,
           Wout, bout, W1, b1, W2, b2):
    src = edge_index[0].astype(jnp.int32).reshape(NW, NCHUNK, CH)
    dst = edge_index[1].astype(jnp.int32).reshape(NW, NCHUNK, CH)
    zero = jnp.zeros((RPT, ME_W), F32)
    lane16 = lax.broadcasted_iota(jnp.int32, (16, 16), 1)
    row16 = lax.broadcasted_iota(jnp.int32, (16, 16), 0)
    cons = ((lane16 == row16) & (row16 < H)).astype(F32) + (
        (row16 == H) & (lane16 < H)).astype(F32)
    sc_fused = _sc_kernels()
    h = _embed(x, W_emb)
    for l in range(2):
        qv, kk = _qkv(h, Wqk[l], Wv[l])
        acc = sc_fused(qv, kk, src, dst, zero, cons)
        h = _ffn(h, acc, Wout[l], bout[l].reshape(1, D), W1[l],
                 b1[l].reshape(1, FF), W2[l], b2[l].reshape(1, D))
    return _pool(h, batch.astype(jnp.int32).reshape(N, 1))
